# baseline (device time: 20779 ns/iter reference)
import jax
import jax.numpy as jnp
from jax import lax
from jax.experimental import pallas as pl
from jax.experimental.pallas import tpu as pltpu

N_DEV = 4


def kernel(x, router_W, route_idx, expert_W, shared_W):
    n_tok, d_in = x.shape
    e_per, _, d_out = expert_W.shape

    def body(x_ref, rw_ref, idx_ref, ew_ref, sw_ref, out_ref,
             comm_ref, send_sems, recv_sems):
        my = lax.axis_index("i")
        left = lax.rem(my + N_DEV - 1, N_DEV)
        right = lax.rem(my + 1, N_DEV)

        barrier_sem = pltpu.get_barrier_semaphore()
        for nbr in (left, right):
            pl.semaphore_signal(
                barrier_sem, inc=1,
                device_id=(nbr,), device_id_type=pl.DeviceIdType.MESH,
            )
        pl.semaphore_wait(barrier_sem, 2)

        xv = x_ref[:, :]

        scores = jnp.dot(xv, rw_ref[:, :], preferred_element_type=jnp.float32)
        s_max = jnp.max(scores, axis=1, keepdims=True)
        p = jnp.exp(scores - s_max)
        probs = p / jnp.sum(p, axis=1, keepdims=True)

        ridx = idx_ref[:, :]
        cols = lax.broadcasted_iota(jnp.int32, probs.shape, 1)
        chosen = jnp.sum(jnp.where(cols == ridx, probs, 0.0),
                         axis=1, keepdims=True)

        partial = jnp.zeros((n_tok, d_out), jnp.float32)
        for le in range(e_per):
            e_id = my * e_per + le
            gate = jnp.where(ridx == e_id, chosen, 0.0)
            xw = jnp.dot(xv, ew_ref[le], preferred_element_type=jnp.float32)
            partial = partial + gate * xw

        comm_ref[0, :, :] = partial
        shared = jnp.dot(xv, sw_ref[:, :], preferred_element_type=jnp.float32)
        out_ref[:, :] = partial + shared

        for h in range(N_DEV - 1):
            rdma = pltpu.make_async_remote_copy(
                src_ref=comm_ref.at[h],
                dst_ref=comm_ref.at[h + 1],
                send_sem=send_sems.at[h],
                recv_sem=recv_sems.at[h],
                device_id=(right,),
                device_id_type=pl.DeviceIdType.MESH,
            )
            rdma.start()
            rdma.wait()
            out_ref[:, :] = out_ref[:, :] + comm_ref[h + 1, :, :]

    return pl.pallas_call(
        body,
        out_shape=jax.ShapeDtypeStruct((n_tok, d_out), jnp.float32),
        in_specs=[pl.BlockSpec(memory_space=pltpu.VMEM)] * 5,
        out_specs=pl.BlockSpec(memory_space=pltpu.VMEM),
        scratch_shapes=[
            pltpu.VMEM((N_DEV, n_tok, d_out), jnp.float32),
            pltpu.SemaphoreType.DMA((N_DEV - 1,)),
            pltpu.SemaphoreType.DMA((N_DEV - 1,)),
        ],
        compiler_params=pltpu.CompilerParams(collective_id=0),
    )(x, router_W, route_idx, expert_W, shared_W)


# device time: 14057 ns/iter; 1.4782x vs baseline; 1.4782x over previous
import jax
import jax.numpy as jnp
from jax import lax
from jax.experimental import pallas as pl
from jax.experimental.pallas import tpu as pltpu

N_DEV = 4


def kernel(x, router_W, route_idx, expert_W, shared_W):
    n_tok, d_in = x.shape
    e_per, _, d_out = expert_W.shape
    chunk = n_tok // N_DEV

    def body(x_ref, rw_ref, idx_ref, ew_ref, sw_ref, out_ref,
             part_ref, rs_buf, ag_buf,
             rs_send_sems, rs_recv_sems, ag_send_sems, ag_recv_sems):
        my = lax.axis_index("i")

        barrier_sem = pltpu.get_barrier_semaphore()
        for d in range(1, N_DEV):
            peer = lax.rem(my + d, N_DEV)
            pl.semaphore_signal(
                barrier_sem, inc=1,
                device_id=(peer,), device_id_type=pl.DeviceIdType.MESH,
            )
        pl.semaphore_wait(barrier_sem, N_DEV - 1)

        xv = x_ref[:, :]

        scores = jnp.dot(xv, rw_ref[:, :], preferred_element_type=jnp.float32)
        s_max = jnp.max(scores, axis=1, keepdims=True)
        p = jnp.exp(scores - s_max)
        probs = p / jnp.sum(p, axis=1, keepdims=True)

        ridx = idx_ref[:, :]
        cols = lax.broadcasted_iota(jnp.int32, probs.shape, 1)
        chosen = jnp.sum(jnp.where(cols == ridx, probs, 0.0),
                         axis=1, keepdims=True)

        partial = jnp.zeros((n_tok, d_out), jnp.float32)
        for le in range(e_per):
            e_id = my * e_per + le
            gate = jnp.where(ridx == e_id, chosen, 0.0)
            xw = jnp.dot(xv, ew_ref[le], preferred_element_type=jnp.float32)
            partial = partial + gate * xw
        part_ref[:, :] = partial

        rs_sends = []
        for d in range(1, N_DEV):
            dest = lax.rem(my + d, N_DEV)
            rdma = pltpu.make_async_remote_copy(
                src_ref=part_ref.at[pl.ds(dest * chunk, chunk), :],
                dst_ref=rs_buf.at[my],
                send_sem=rs_send_sems.at[dest],
                recv_sem=rs_recv_sems.at[my],
                device_id=(dest,),
                device_id_type=pl.DeviceIdType.MESH,
            )
            rdma.start()
            rs_sends.append(rdma)

        shared = jnp.dot(xv, sw_ref[:, :], preferred_element_type=jnp.float32)
        out_ref[:, :] = shared

        red = part_ref[pl.ds(my * chunk, chunk), :]
        for d in range(1, N_DEV):
            src = lax.rem(my + d, N_DEV)
            recv = pltpu.make_async_remote_copy(
                src_ref=part_ref.at[pl.ds(0, chunk), :],
                dst_ref=rs_buf.at[src],
                send_sem=rs_send_sems.at[src],
                recv_sem=rs_recv_sems.at[src],
                device_id=(src,),
                device_id_type=pl.DeviceIdType.MESH,
            )
            recv.wait_recv()
            red = red + rs_buf[src, :, :]
        ag_buf[my, :, :] = red

        ag_sends = []
        for d in range(1, N_DEV):
            dest = lax.rem(my + d, N_DEV)
            rdma = pltpu.make_async_remote_copy(
                src_ref=ag_buf.at[my],
                dst_ref=ag_buf.at[my],
                send_sem=ag_send_sems.at[dest],
                recv_sem=ag_recv_sems.at[my],
                device_id=(dest,),
                device_id_type=pl.DeviceIdType.MESH,
            )
            rdma.start()
            ag_sends.append(rdma)

        out_ref[pl.ds(my * chunk, chunk), :] = (
            out_ref[pl.ds(my * chunk, chunk), :] + red
        )

        for d in range(1, N_DEV):
            src = lax.rem(my + d, N_DEV)
            recv = pltpu.make_async_remote_copy(
                src_ref=ag_buf.at[my],
                dst_ref=ag_buf.at[src],
                send_sem=ag_send_sems.at[src],
                recv_sem=ag_recv_sems.at[src],
                device_id=(src,),
                device_id_type=pl.DeviceIdType.MESH,
            )
            recv.wait_recv()
            out_ref[pl.ds(src * chunk, chunk), :] = (
                out_ref[pl.ds(src * chunk, chunk), :] + ag_buf[src, :, :]
            )

        for rdma in rs_sends + ag_sends:
            rdma.wait_send()

    return pl.pallas_call(
        body,
        out_shape=jax.ShapeDtypeStruct((n_tok, d_out), jnp.float32),
        in_specs=[pl.BlockSpec(memory_space=pltpu.VMEM)] * 5,
        out_specs=pl.BlockSpec(memory_space=pltpu.VMEM),
        scratch_shapes=[
            pltpu.VMEM((n_tok, d_out), jnp.float32),
            pltpu.VMEM((N_DEV, chunk, d_out), jnp.float32),
            pltpu.VMEM((N_DEV, chunk, d_out), jnp.float32),
            pltpu.SemaphoreType.DMA((N_DEV,)),
            pltpu.SemaphoreType.DMA((N_DEV,)),
            pltpu.SemaphoreType.DMA((N_DEV,)),
            pltpu.SemaphoreType.DMA((N_DEV,)),
        ],
        compiler_params=pltpu.CompilerParams(collective_id=0),
    )(x, router_W, route_idx, expert_W, shared_W)
